# baseline (speedup 1.0000x reference)
import functools

import jax
import jax.numpy as jnp
from jax import lax
from jax.experimental import pallas as pl
from jax.experimental.pallas import tpu as pltpu
from jax.experimental.pallas import tpu_sc as plsc

_L = 16
_NBUF = 4
_UNROLL = 10


@functools.lru_cache(maxsize=None)
def _build_fm(B, F, D, V, NC, NS):
    NW = NC * NS
    RPW = B // NW
    FP = ((F + _L - 1) // _L) * _L
    NJ = D // _L
    CH = ((V + NS - 1) // NS + 7) // 8 * 8
    CH_LAST = V - CH * (NS - 1)
    assert CH_LAST > 0

    mesh = plsc.VectorSubcoreMesh(core_axis_name="c", subcore_axis_name="s",
                                  num_cores=NC, num_subcores=NS)

    def body(x_hbm, emb_hbm, proj_hbm, fcw_hbm, fcb_hbm, out_hbm,
             idx_v, lin_v, fc_v, proj_sh, bounce, *ring):
        bufs = ring[:_NBUF]
        pbufs = ring[_NBUF:2 * _NBUF]
        sems_e = ring[2 * _NBUF:3 * _NBUF]
        sems_p = ring[3 * _NBUF:4 * _NBUF]

        sid = lax.axis_index("s")
        wid = sid * NC + lax.axis_index("c")
        base = wid * RPW

        cx = pltpu.make_async_copy(x_hbm.at[pl.ds(base, RPW)], idx_v,
                                   sems_e[0])
        cx.start()
        cw = pltpu.make_async_copy(fcw_hbm, fc_v.at[pl.ds(0, 1)], sems_e[1])
        cw.start()
        cb = pltpu.make_async_copy(fcb_hbm, fc_v.at[pl.ds(8, 1)], sems_e[2])
        cb.start()

        @pl.when(sid < NS - 1)
        def _stage():
            pltpu.sync_copy(proj_hbm.at[pl.ds(sid * CH, CH)], bounce)
            pltpu.sync_copy(bounce, proj_sh.at[pl.ds(sid * CH, CH)])

        @pl.when(sid == NS - 1)
        def _stage_last():
            pltpu.sync_copy(proj_hbm.at[pl.ds((NS - 1) * CH, CH_LAST)],
                            bounce.at[pl.ds(0, CH_LAST)])
            pltpu.sync_copy(bounce.at[pl.ds(0, CH_LAST)],
                            proj_sh.at[pl.ds((NS - 1) * CH, CH_LAST)])

        cw.wait()
        cb.wait()
        cx.wait()

        zeros = jnp.zeros((_L,), jnp.float32)
        for pb in pbufs:
            pb[pl.ds(FP - _L, _L)] = zeros

        def copies(b, k):
            idx_row = idx_v.at[b]
            return (pltpu.make_async_copy(emb_hbm.at[idx_row], bufs[k], sems_e[k]),
                    pltpu.make_async_copy(proj_sh.at[idx_row],
                                          pbufs[k].at[pl.ds(0, F)], sems_p[k]))

        def start(b, k):
            ce, cp = copies(b, k)
            ce.start()
            cp.start()

        def wait(b, k):
            ce, cp = copies(b, k)
            ce.wait()
            cp.wait()

        def process(b, k):
            buf, pbuf = bufs[k], pbufs[k]

            init = tuple(jnp.zeros((_L,), jnp.float32) for _ in range(2 * NJ))

            @plsc.parallel_loop(0, F, step=1, unroll=_UNROLL, carry=init)
            def accs(r, carry):
                new_s, new_q = [], []
                for j in range(NJ):
                    v = buf[r, pl.ds(j * _L, _L)]
                    new_s.append(carry[j] + v)
                    new_q.append(carry[NJ + j] + v * v)
                return tuple(new_s) + tuple(new_q)
            u = accs[0] * accs[0]
            q = accs[NJ]
            for j in range(1, NJ):
                u = u + accs[j] * accs[j]
                q = q + accs[NJ + j]
            p = pbuf[pl.ds(0, _L)]
            for j in range(1, FP // _L):
                p = p + pbuf[pl.ds(j * _L, _L)]
            lin = jnp.sum(p + 0.5 * (u - q))
            idxv = jnp.full((_L,), b, dtype=jnp.int32)
            valv = jnp.full((_L,), lin, dtype=jnp.float32)
            mask = lax.iota(jnp.int32, _L) == 0
            plsc.store_scatter(lin_v, [idxv], valv, mask=mask)

        for k in range(_NBUF - 1):
            copies(k, k)[0].start()
        plsc.subcore_barrier()
        for k in range(_NBUF - 1):
            copies(k, k)[1].start()

        def outer(i, _):
            for k in range(_NBUF):
                b = _NBUF * i + k

                @pl.when(b + _NBUF - 1 < RPW)
                def _start():
                    start(b + _NBUF - 1, (k + _NBUF - 1) % _NBUF)

                wait(b, k)
                process(b, k)
            return _

        lax.fori_loop(0, RPW // _NBUF, outer, 0)

        fcvec = fc_v[...]
        fcw = fcvec[0]
        fcb = fcvec[8]
        for j in range(RPW // _L):
            v = lin_v[pl.ds(j * _L, _L)]
            logit = v * fcw + fcb
            lin_v[pl.ds(j * _L, _L)] = 1.0 / (1.0 + jnp.exp(-logit))
        pltpu.sync_copy(lin_v, out_hbm.at[pl.ds(base, RPW)])

    return pl.kernel(
        body,
        out_type=jax.ShapeDtypeStruct((B,), jnp.float32),
        mesh=mesh,
        compiler_params=pltpu.CompilerParams(needs_layout_passes=False),
        scratch_types=(
            [
                pltpu.VMEM((RPW, F), jnp.int32),
                pltpu.VMEM((RPW,), jnp.float32),
                pltpu.VMEM((_L,), jnp.float32),
                pltpu.VMEM_SHARED((V,), jnp.float32),
                pltpu.VMEM((CH,), jnp.float32),
            ]
            + [pltpu.VMEM((F, D), jnp.float32) for _ in range(_NBUF)]
            + [pltpu.VMEM((FP,), jnp.float32) for _ in range(_NBUF)]
            + [pltpu.SemaphoreType.DMA for _ in range(2 * _NBUF)]
        ),
    )


def kernel(x, emb, proj_w, fc_w, fc_b):
    B, F = x.shape
    D = emb.shape[1]
    info = plsc.get_sparse_core_info()
    fm = _build_fm(B, F, D, emb.shape[0], info.num_cores, info.num_subcores)
    if x.dtype != jnp.int32:
        x = x.astype(jnp.int32)
    return fm(x, emb, proj_w.reshape(-1), fc_w.reshape(-1), fc_b)


# speedup vs baseline: 1.0019x; 1.0019x over previous
import functools

import jax
import jax.numpy as jnp
from jax import lax
from jax.experimental import pallas as pl
from jax.experimental.pallas import tpu as pltpu
from jax.experimental.pallas import tpu_sc as plsc

_L = 16
_NBUF = 4
_UNROLL = 4


@functools.lru_cache(maxsize=None)
def _build_fm(B, F, D, V, NC, NS):
    NW = NC * NS
    RPW = B // NW
    FP = ((F + _L - 1) // _L) * _L
    NJ = D // _L
    CH = ((V + NS - 1) // NS + 7) // 8 * 8
    CH_LAST = V - CH * (NS - 1)
    assert CH_LAST > 0

    mesh = plsc.VectorSubcoreMesh(core_axis_name="c", subcore_axis_name="s",
                                  num_cores=NC, num_subcores=NS)

    def body(x_hbm, emb_hbm, proj_hbm, fcw_hbm, fcb_hbm, out_hbm,
             idx_v, lin_v, fc_v, proj_sh, bounce, *ring):
        bufs = ring[:_NBUF]
        pbufs = ring[_NBUF:2 * _NBUF]
        sems_e = ring[2 * _NBUF:3 * _NBUF]
        sems_p = ring[3 * _NBUF:4 * _NBUF]

        sid = lax.axis_index("s")
        wid = sid * NC + lax.axis_index("c")
        base = wid * RPW

        cx = pltpu.make_async_copy(x_hbm.at[pl.ds(base, RPW)], idx_v,
                                   sems_e[0])
        cx.start()
        cw = pltpu.make_async_copy(fcw_hbm, fc_v.at[pl.ds(0, 1)], sems_e[1])
        cw.start()
        cb = pltpu.make_async_copy(fcb_hbm, fc_v.at[pl.ds(8, 1)], sems_e[2])
        cb.start()

        @pl.when(sid < NS - 1)
        def _stage():
            pltpu.sync_copy(proj_hbm.at[pl.ds(sid * CH, CH)], bounce)
            pltpu.sync_copy(bounce, proj_sh.at[pl.ds(sid * CH, CH)])

        @pl.when(sid == NS - 1)
        def _stage_last():
            pltpu.sync_copy(proj_hbm.at[pl.ds((NS - 1) * CH, CH_LAST)],
                            bounce.at[pl.ds(0, CH_LAST)])
            pltpu.sync_copy(bounce.at[pl.ds(0, CH_LAST)],
                            proj_sh.at[pl.ds((NS - 1) * CH, CH_LAST)])

        cw.wait()
        cb.wait()
        cx.wait()

        zeros = jnp.zeros((_L,), jnp.float32)
        for pb in pbufs:
            pb[pl.ds(FP - _L, _L)] = zeros

        def copies(b, k):
            idx_row = idx_v.at[b]
            return (pltpu.make_async_copy(emb_hbm.at[idx_row], bufs[k], sems_e[k]),
                    pltpu.make_async_copy(proj_sh.at[idx_row],
                                          pbufs[k].at[pl.ds(0, F)], sems_p[k]))

        def start(b, k):
            ce, cp = copies(b, k)
            ce.start()
            cp.start()

        def wait(b, k):
            ce, cp = copies(b, k)
            ce.wait()
            cp.wait()

        def process(b, k):
            buf, pbuf = bufs[k], pbufs[k]

            init = tuple(jnp.zeros((_L,), jnp.float32) for _ in range(2 * NJ))

            @plsc.parallel_loop(0, F, step=1, unroll=_UNROLL, carry=init)
            def accs(r, carry):
                new_s, new_q = [], []
                for j in range(NJ):
                    v = buf[r, pl.ds(j * _L, _L)]
                    new_s.append(carry[j] + v)
                    new_q.append(carry[NJ + j] + v * v)
                return tuple(new_s) + tuple(new_q)
            u = accs[0] * accs[0]
            q = accs[NJ]
            for j in range(1, NJ):
                u = u + accs[j] * accs[j]
                q = q + accs[NJ + j]
            p = pbuf[pl.ds(0, _L)]
            for j in range(1, FP // _L):
                p = p + pbuf[pl.ds(j * _L, _L)]
            lin = jnp.sum(p + 0.5 * (u - q))
            idxv = jnp.full((_L,), b, dtype=jnp.int32)
            valv = jnp.full((_L,), lin, dtype=jnp.float32)
            mask = lax.iota(jnp.int32, _L) == 0
            plsc.store_scatter(lin_v, [idxv], valv, mask=mask)

        for k in range(_NBUF - 1):
            copies(k, k)[0].start()
        plsc.subcore_barrier()
        for k in range(_NBUF - 1):
            copies(k, k)[1].start()

        def outer(i, _):
            for k in range(_NBUF):
                b = _NBUF * i + k

                @pl.when(b + _NBUF - 1 < RPW)
                def _start():
                    start(b + _NBUF - 1, (k + _NBUF - 1) % _NBUF)

                wait(b, k)
                process(b, k)
            return _

        lax.fori_loop(0, RPW // _NBUF, outer, 0)

        fcvec = fc_v[...]
        fcw = fcvec[0]
        fcb = fcvec[8]
        for j in range(RPW // _L):
            v = lin_v[pl.ds(j * _L, _L)]
            logit = v * fcw + fcb
            lin_v[pl.ds(j * _L, _L)] = 1.0 / (1.0 + jnp.exp(-logit))
        pltpu.sync_copy(lin_v, out_hbm.at[pl.ds(base, RPW)])

    return pl.kernel(
        body,
        out_type=jax.ShapeDtypeStruct((B,), jnp.float32),
        mesh=mesh,
        compiler_params=pltpu.CompilerParams(needs_layout_passes=False),
        scratch_types=(
            [
                pltpu.VMEM((RPW, F), jnp.int32),
                pltpu.VMEM((RPW,), jnp.float32),
                pltpu.VMEM((_L,), jnp.float32),
                pltpu.VMEM_SHARED((V,), jnp.float32),
                pltpu.VMEM((CH,), jnp.float32),
            ]
            + [pltpu.VMEM((F, D), jnp.float32) for _ in range(_NBUF)]
            + [pltpu.VMEM((FP,), jnp.float32) for _ in range(_NBUF)]
            + [pltpu.SemaphoreType.DMA for _ in range(2 * _NBUF)]
        ),
    )


def kernel(x, emb, proj_w, fc_w, fc_b):
    B, F = x.shape
    D = emb.shape[1]
    info = plsc.get_sparse_core_info()
    fm = _build_fm(B, F, D, emb.shape[0], info.num_cores, info.num_subcores)
    if x.dtype != jnp.int32:
        x = x.astype(jnp.int32)
    return fm(x, emb, proj_w.reshape(-1), fc_w.reshape(-1), fc_b)
